# baseline (speedup 1.0000x reference)
import functools

import jax
import jax.numpy as jnp
from jax import lax
from jax.experimental import pallas as pl
from jax.experimental.pallas import tpu as pltpu
from jax.experimental.pallas import tpu_sc as plsc

N, E, D, H, O, B = 10000, 320000, 128, 128, 128, 8192
NC, NS, LANES = 2, 16, 16
K = 80
EB = 25
NBLK = E // NS // K // EB
NPAD = 10240
DPT = NPAD // NS
APT = N // NS
IC, IK = 4, 128
RB = 2000
GR = N // RB
CPT = B // (NC * NS)

_mesh = plsc.VectorSubcoreMesh(core_axis_name="c", subcore_axis_name="s")


@functools.partial(
    pl.kernel,
    out_type=(
        jax.ShapeDtypeStruct((NC, N, D), jnp.float32),
        jax.ShapeDtypeStruct((NC, NPAD), jnp.float32),
        jax.ShapeDtypeStruct((NC, NS, IC, IK), jnp.int32),
    ),
    mesh=_mesh,
    scratch_types=[
        pltpu.VMEM((K, D), jnp.float32),
        pltpu.VMEM((K, D), jnp.float32),
        pltpu.VMEM((K, D), jnp.float32),
        pltpu.VMEM((2, EB, K), jnp.int32),
        pltpu.VMEM((IC, IK), jnp.int32),
        pltpu.VMEM((IC, IK), jnp.int32),
        pltpu.VMEM((IC * IK,), jnp.float32),
        pltpu.VMEM((DPT,), jnp.float32),
        pltpu.VMEM((IK,), jnp.float32),
        pltpu.VMEM_SHARED((N, D), jnp.float32),
        pltpu.VMEM_SHARED((NPAD,), jnp.float32),
        pltpu.VMEM_SHARED((NPAD,), jnp.float32),
        pltpu.SemaphoreType.DMA,
        pltpu.SemaphoreType.DMA,
        pltpu.SemaphoreType.DMA,
        pltpu.SemaphoreType.DMA,
        pltpu.SemaphoreType.DMA,
        pltpu.SemaphoreType.DMA,
        pltpu.SemaphoreType.DMA,
    ],
    compiler_params=pltpu.CompilerParams(needs_layout_passes=False),
)
def _stage_a(x2, edges, ids,
             agg_out, deg_out, idx_out,
             buf0, buf1, buf2, ebuf, ids_vm, idx_vm, rk_vm,
             zero1d, ones_vm, agg_sp, deg_sp, pres_sp,
             semG0, semG1, semG2, semS0, semS1, semS2, semD):
    cid = lax.axis_index("c")
    sid = lax.axis_index("s")
    dbase = sid * DPT
    abase = sid * APT
    bufs = (buf0, buf1, buf2)
    gsems = (semG0, semG1, semG2)
    ssems = (semS0, semS1, semS2)

    z16 = jnp.zeros((LANES,), jnp.float32)
    o16 = jnp.ones((LANES,), jnp.float32)

    def _zrow(r, carry):
        for c in range(D // LANES):
            buf0[r, pl.ds(c * LANES, LANES)] = z16
        return carry
    lax.fori_loop(0, K, _zrow, 0)

    def _z1d(k, carry):
        zero1d[pl.ds(k * LANES, LANES)] = z16
        return carry
    lax.fori_loop(0, DPT // LANES, _z1d, 0)

    for c in range(IK // LANES):
        ones_vm[pl.ds(c * LANES, LANES)] = o16

    for z in range(7):
        pltpu.sync_copy(buf0, agg_sp.at[pl.ds(abase + z * K, K)])
    pltpu.sync_copy(buf0.at[pl.ds(0, APT - 7 * K)],
                    agg_sp.at[pl.ds(abase + 7 * K, APT - 7 * K)])
    pltpu.sync_copy(zero1d, deg_sp.at[pl.ds(dbase, DPT)])
    pltpu.sync_copy(zero1d, pres_sp.at[pl.ds(dbase, DPT)])

    pltpu.sync_copy(ids.at[cid, sid], ids_vm)

    plsc.subcore_barrier()

    def _eblk(b, carry):
        pltpu.sync_copy(edges.at[cid, sid, b], ebuf)
        for c in range(3):
            pltpu.async_copy(x2.at[ebuf.at[0, c]], bufs[c], gsems[c])
        for c in range(EB):
            i3 = c % 3
            pltpu.make_async_copy(x2.at[ebuf.at[0, c]], bufs[i3],
                                  gsems[i3]).wait()
            pltpu.async_copy(bufs[i3], agg_sp.at[ebuf.at[1, c]], ssems[i3],
                             add=True)
            pltpu.async_copy(ones_vm.at[pl.ds(0, K)],
                             deg_sp.at[ebuf.at[1, c]], semD, add=True)
            if c >= 1 and c + 2 < EB:
                j3 = (c - 1) % 3
                pltpu.make_async_copy(bufs[j3], agg_sp.at[ebuf.at[1, c - 1]],
                                      ssems[j3]).wait()
                pltpu.async_copy(x2.at[ebuf.at[0, c + 2]], bufs[j3],
                                 gsems[j3])
        for c in (EB - 3, EB - 2, EB - 1):
            pltpu.make_async_copy(bufs[c % 3], agg_sp.at[ebuf.at[1, c]],
                                  ssems[c % 3]).wait()
        for c in range(EB):
            pltpu.make_async_copy(ones_vm.at[pl.ds(0, K)],
                                  deg_sp.at[ebuf.at[1, c]], semD).wait()
        return carry
    lax.fori_loop(0, NBLK, _eblk, 0)

    for ci in range(IC):
        pltpu.sync_copy(ones_vm, pres_sp.at[ids_vm.at[ci]], add=True)

    plsc.subcore_barrier()

    @pl.when(sid == 0)
    def _():
        def _chunk(c, carry):
            pltpu.sync_copy(pres_sp.at[pl.ds(c * DPT, DPT)], zero1d)

            def _pbody(k, cin):
                v = zero1d[pl.ds(k * LANES, LANES)]
                p1 = jnp.minimum(v, 1.0)
                cs = plsc.cumsum(p1)
                zero1d[pl.ds(k * LANES, LANES)] = cin + cs - p1
                return cin + cs[LANES - 1]
            c2 = lax.fori_loop(0, DPT // LANES, _pbody, carry)
            pltpu.sync_copy(zero1d, pres_sp.at[pl.ds(c * DPT, DPT)])
            return c2
        lax.fori_loop(0, NS, _chunk, jnp.float32(0.0))

    _nz = N // K // NS + 1
    for z in range(_nz):
        cix = z * NS + sid

        @pl.when(cix < N // K)
        def _(z=z, cix=cix):
            i2 = z % 2
            if z >= 2:
                pc = (z - 2) * NS + sid
                pltpu.make_async_copy(bufs[i2],
                                      agg_out.at[cid, pl.ds(pc * K, K)],
                                      ssems[i2]).wait()
            pltpu.async_copy(agg_sp.at[pl.ds(cix * K, K)], bufs[i2],
                             gsems[i2]).wait()
            pltpu.async_copy(bufs[i2], agg_out.at[cid, pl.ds(cix * K, K)],
                             ssems[i2])
    for z in (_nz - 2, _nz - 1):
        cix = z * NS + sid

        @pl.when(cix < N // K)
        def _(z=z, cix=cix):
            pltpu.make_async_copy(bufs[z % 2],
                                  agg_out.at[cid, pl.ds(cix * K, K)],
                                  ssems[z % 2]).wait()
    pltpu.sync_copy(deg_sp.at[pl.ds(dbase, DPT)], zero1d)
    pltpu.sync_copy(zero1d, deg_out.at[cid, pl.ds(dbase, DPT)])

    plsc.subcore_barrier()

    for ci in range(IC):
        pltpu.async_copy(pres_sp.at[ids_vm.at[ci]],
                         rk_vm.at[pl.ds(ci * IK, IK)], semG0).wait()
    off = cid * N
    for ci in range(IC):
        for s8 in range(IK // LANES):
            rv = rk_vm[pl.ds(ci * IK + s8 * LANES, LANES)]
            idx_vm[ci, pl.ds(s8 * LANES, LANES)] = rv.astype(jnp.int32) + off
    pltpu.sync_copy(idx_vm, idx_out.at[cid, sid])


def _tc_body1(x_ref, ws_ref, bs_ref, bn_ref, t_ref):
    t_ref[...] = (jnp.dot(x_ref[...], ws_ref[...],
                          preferred_element_type=jnp.float32)
                  + bs_ref[...] + bn_ref[...])


def _stage_b1(x2, w_self, bs, bn):
    return pl.pallas_call(
        _tc_body1,
        grid=(NC * GR,),
        in_specs=[
            pl.BlockSpec((RB, D), lambda r: (r, 0)),
            pl.BlockSpec((D, H), lambda r: (0, 0)),
            pl.BlockSpec((1, H), lambda r: (0, 0)),
            pl.BlockSpec((1, H), lambda r: (0, 0)),
        ],
        out_specs=pl.BlockSpec((RB, H), lambda r: (r, 0)),
        out_shape=jax.ShapeDtypeStruct((NC * N, H), jnp.float32),
    )(x2, w_self, bs, bn)


def _tc_body2(t_ref, agg_ref, deg_ref, wn_ref, wh_ref, p_ref):
    a = agg_ref[0]
    dg = deg_ref[0, 0].reshape(RB, 1)
    mean = a / jnp.maximum(dg, 1.0)
    t = t_ref[...] + jnp.dot(mean, wn_ref[...],
                             preferred_element_type=jnp.float32)
    h = jnp.maximum(t, 0.0)
    nrm = jnp.sqrt(jnp.sum(h * h, axis=1, keepdims=True))
    s = h / jnp.maximum(nrm, 1e-12)
    p_ref[0] = jnp.dot(s, wh_ref[0], preferred_element_type=jnp.float32)


def _stage_b2(t, agg, deg3, w_neigh, wh):
    return pl.pallas_call(
        _tc_body2,
        grid=(NC, GR),
        in_specs=[
            pl.BlockSpec((RB, H), lambda g, r: (g * GR + r, 0)),
            pl.BlockSpec((1, RB, D), lambda g, r: (g, r, 0)),
            pl.BlockSpec((1, 1, RB), lambda g, r: (g * GR + r, 0, 0)),
            pl.BlockSpec((D, H), lambda g, r: (0, 0)),
            pl.BlockSpec((1, H, O), lambda g, r: (g, 0, 0)),
        ],
        out_specs=pl.BlockSpec((1, RB, O), lambda g, r: (g, r, 0)),
        out_shape=jax.ShapeDtypeStruct((NC, N, O), jnp.float32),
    )(t, agg, deg3, w_neigh, wh)


@functools.partial(
    pl.kernel,
    out_type=jax.ShapeDtypeStruct((B, O), jnp.float32),
    mesh=_mesh,
    scratch_types=[
        pltpu.VMEM((CPT // IK, IK), jnp.int32),
        pltpu.VMEM((CPT // IK, IK), jnp.int32),
        pltpu.VMEM((CPT, O), jnp.float32),
        pltpu.VMEM((CPT, O), jnp.float32),
        pltpu.VMEM((O,), jnp.float32),
        pltpu.SemaphoreType.DMA,
    ],
    compiler_params=pltpu.CompilerParams(needs_layout_passes=False),
)
def _stage_c(p2, idxr, b_out, out, idxA, idxB, rowsA, rowsB, bvm, sem):
    cid = lax.axis_index("c")
    sid = lax.axis_index("s")
    w = cid * NS + sid

    pltpu.sync_copy(idxr.at[0, w], idxA)
    pltpu.sync_copy(idxr.at[1, w], idxB)
    pltpu.sync_copy(b_out, bvm)
    for cc in range(CPT // IK):
        pltpu.async_copy(p2.at[idxA.at[cc]],
                         rowsA.at[pl.ds(cc * IK, IK)], sem).wait()
        pltpu.async_copy(p2.at[idxB.at[cc]],
                         rowsB.at[pl.ds(cc * IK, IK)], sem).wait()

    def _rbody(r, carry):
        for c8 in range(O // LANES):
            sl = pl.ds(c8 * LANES, LANES)
            rowsA[r, sl] = rowsA[r, sl] + rowsB[r, sl] + bvm[sl]
        return carry
    lax.fori_loop(0, CPT, _rbody, 0)

    pltpu.sync_copy(rowsA, out.at[pl.ds(w * CPT, CPT)])


def kernel(x_i, edge_index_i, node_ids_i, x_j, edge_index_j, node_ids_j,
           W_self, b_self, W_neigh, b_neigh, W_out, b_out):
    x_i = x_i.astype(jnp.float32)
    x_j = x_j.astype(jnp.float32)
    x2 = jnp.concatenate([x_i, x_j], axis=0)
    ei = edge_index_i.astype(jnp.int32)
    ej = edge_index_j.astype(jnp.int32)
    e_i = ei.reshape(2, NS, NBLK, EB, K)
    e_j = jnp.stack([ej[0] + N, ej[1]]).reshape(2, NS, NBLK, EB, K)
    edges = jnp.stack([e_i, e_j]).transpose(0, 2, 3, 1, 4, 5)
    ids = jnp.stack([node_ids_i.astype(jnp.int32).reshape(NS, IC, IK),
                     node_ids_j.astype(jnp.int32).reshape(NS, IC, IK)])

    bs = b_self.astype(jnp.float32).reshape(1, H)
    bn = b_neigh.astype(jnp.float32).reshape(1, H)
    wh = W_out.astype(jnp.float32).reshape(NC, H, O)
    t = _stage_b1(x2, W_self.astype(jnp.float32), bs, bn)
    agg, deg, idx = _stage_a(x2, edges, ids)
    deg3 = deg[:, :N].reshape(NC * GR, 1, RB)
    p = _stage_b2(t, agg, deg3, W_neigh.astype(jnp.float32), wh)

    p2 = p.reshape(NC * N, O)
    idxr = idx.reshape(NC, NC * NS, CPT // IK, IK)
    h = _stage_c(p2, idxr, b_out.astype(jnp.float32))
    return h[None]


# speedup vs baseline: 1.0562x; 1.0562x over previous
import functools

import jax
import jax.numpy as jnp
from jax import lax
from jax.experimental import pallas as pl
from jax.experimental.pallas import tpu as pltpu
from jax.experimental.pallas import tpu_sc as plsc

N, E, D, H, O, B = 10000, 320000, 128, 128, 128, 8192
NC, NS, LANES = 2, 16, 16
K = 80
EB = 25
NBLK = E // NS // K // EB
NPAD = 10240
DPT = NPAD // NS
APT = N // NS
IC, IK = 4, 128
RB = 2000
GR = N // RB
CPT = B // (NC * NS)

_mesh = plsc.VectorSubcoreMesh(core_axis_name="c", subcore_axis_name="s")


@functools.partial(
    pl.kernel,
    out_type=(
        jax.ShapeDtypeStruct((NC, N, D), jnp.float32),
        jax.ShapeDtypeStruct((NC, NPAD), jnp.float32),
        jax.ShapeDtypeStruct((NC, NS, IC, IK), jnp.int32),
    ),
    mesh=_mesh,
    scratch_types=[
        pltpu.VMEM((K, D), jnp.float32),
        pltpu.VMEM((K, D), jnp.float32),
        pltpu.VMEM((K, D), jnp.float32),
        pltpu.VMEM((2, EB, K), jnp.int32),
        pltpu.VMEM((IC, IK), jnp.int32),
        pltpu.VMEM((IC, IK), jnp.int32),
        pltpu.VMEM((IC * IK,), jnp.float32),
        pltpu.VMEM((DPT,), jnp.float32),
        pltpu.VMEM((IK,), jnp.float32),
        pltpu.VMEM_SHARED((N, D), jnp.float32),
        pltpu.VMEM_SHARED((NPAD,), jnp.float32),
        pltpu.VMEM_SHARED((NPAD,), jnp.float32),
        pltpu.VMEM_SHARED((NS * LANES,), jnp.float32),
        pltpu.SemaphoreType.DMA,
        pltpu.SemaphoreType.DMA,
        pltpu.SemaphoreType.DMA,
        pltpu.SemaphoreType.DMA,
        pltpu.SemaphoreType.DMA,
        pltpu.SemaphoreType.DMA,
        pltpu.SemaphoreType.DMA,
    ],
    compiler_params=pltpu.CompilerParams(needs_layout_passes=False),
)
def _stage_a(x2, edges, ids,
             agg_out, deg_out, idx_out,
             buf0, buf1, buf2, ebuf, ids_vm, idx_vm, rk_vm,
             zero1d, ones_vm, agg_sp, deg_sp, pres_sp, tot_sp,
             semG0, semG1, semG2, semS0, semS1, semS2, semD):
    cid = lax.axis_index("c")
    sid = lax.axis_index("s")
    dbase = sid * DPT
    abase = sid * APT
    bufs = (buf0, buf1, buf2)
    gsems = (semG0, semG1, semG2)
    ssems = (semS0, semS1, semS2)

    z16 = jnp.zeros((LANES,), jnp.float32)
    o16 = jnp.ones((LANES,), jnp.float32)

    def _zrow(r, carry):
        for c in range(D // LANES):
            buf0[r, pl.ds(c * LANES, LANES)] = z16
        return carry
    lax.fori_loop(0, K, _zrow, 0)

    def _z1d(k, carry):
        zero1d[pl.ds(k * LANES, LANES)] = z16
        return carry
    lax.fori_loop(0, DPT // LANES, _z1d, 0)

    for c in range(IK // LANES):
        ones_vm[pl.ds(c * LANES, LANES)] = o16

    for z in range(7):
        pltpu.sync_copy(buf0, agg_sp.at[pl.ds(abase + z * K, K)])
    pltpu.sync_copy(buf0.at[pl.ds(0, APT - 7 * K)],
                    agg_sp.at[pl.ds(abase + 7 * K, APT - 7 * K)])
    pltpu.sync_copy(zero1d, deg_sp.at[pl.ds(dbase, DPT)])
    pltpu.sync_copy(zero1d, pres_sp.at[pl.ds(dbase, DPT)])

    pltpu.sync_copy(ids.at[cid, sid], ids_vm)

    plsc.subcore_barrier()

    for ci in range(IC):
        pltpu.sync_copy(ones_vm, pres_sp.at[ids_vm.at[ci]], add=True)

    plsc.subcore_barrier()

    def _eblk(b, carry):
        pltpu.sync_copy(edges.at[cid, sid, b], ebuf)
        for c in range(3):
            pltpu.async_copy(x2.at[ebuf.at[0, c]], bufs[c], gsems[c])
        for c in range(EB):
            i3 = c % 3
            pltpu.make_async_copy(x2.at[ebuf.at[0, c]], bufs[i3],
                                  gsems[i3]).wait()
            pltpu.async_copy(bufs[i3], agg_sp.at[ebuf.at[1, c]], ssems[i3],
                             add=True)
            pltpu.async_copy(ones_vm.at[pl.ds(0, K)],
                             deg_sp.at[ebuf.at[1, c]], semD, add=True)
            if c >= 1 and c + 2 < EB:
                j3 = (c - 1) % 3
                pltpu.make_async_copy(bufs[j3], agg_sp.at[ebuf.at[1, c - 1]],
                                      ssems[j3]).wait()
                pltpu.async_copy(x2.at[ebuf.at[0, c + 2]], bufs[j3],
                                 gsems[j3])
        for c in (EB - 3, EB - 2, EB - 1):
            pltpu.make_async_copy(bufs[c % 3], agg_sp.at[ebuf.at[1, c]],
                                  ssems[c % 3]).wait()
        for c in range(EB):
            pltpu.make_async_copy(ones_vm.at[pl.ds(0, K)],
                                  deg_sp.at[ebuf.at[1, c]], semD).wait()
        return carry
    lax.fori_loop(0, NBLK, _eblk, 0)

    pltpu.sync_copy(pres_sp.at[pl.ds(dbase, DPT)], zero1d)

    def _pbody(k, cin):
        v = zero1d[pl.ds(k * LANES, LANES)]
        p1 = jnp.minimum(v, 1.0)
        cs = plsc.cumsum(p1)
        zero1d[pl.ds(k * LANES, LANES)] = cin + cs - p1
        return cin + cs[LANES - 1]
    total = lax.fori_loop(0, DPT // LANES, _pbody, jnp.float32(0.0))
    rk_vm[pl.ds(0, LANES)] = jnp.zeros((LANES,), jnp.float32) + total
    pltpu.sync_copy(rk_vm.at[pl.ds(0, LANES)],
                    tot_sp.at[pl.ds(sid * LANES, LANES)])

    plsc.subcore_barrier()

    pltpu.sync_copy(tot_sp, rk_vm.at[pl.ds(0, NS * LANES)])
    lane = lax.iota(jnp.int32, LANES)
    tots = plsc.load_gather(rk_vm, [lane * LANES])
    ex = plsc.cumsum(tots) - tots
    sel = jnp.where(lane == sid, ex, 0.0)
    off = plsc.cumsum(sel)[LANES - 1]

    def _obody(k, carry):
        sl = pl.ds(k * LANES, LANES)
        zero1d[sl] = zero1d[sl] + off
        return carry
    lax.fori_loop(0, DPT // LANES, _obody, 0)
    pltpu.sync_copy(zero1d, pres_sp.at[pl.ds(dbase, DPT)])

    _nz = N // K // NS + 1
    for z in range(_nz):
        cix = z * NS + sid

        @pl.when(cix < N // K)
        def _(z=z, cix=cix):
            i2 = z % 2
            if z >= 2:
                pc = (z - 2) * NS + sid
                pltpu.make_async_copy(bufs[i2],
                                      agg_out.at[cid, pl.ds(pc * K, K)],
                                      ssems[i2]).wait()
            pltpu.async_copy(agg_sp.at[pl.ds(cix * K, K)], bufs[i2],
                             gsems[i2]).wait()
            pltpu.async_copy(bufs[i2], agg_out.at[cid, pl.ds(cix * K, K)],
                             ssems[i2])
    for z in (_nz - 2, _nz - 1):
        cix = z * NS + sid

        @pl.when(cix < N // K)
        def _(z=z, cix=cix):
            pltpu.make_async_copy(bufs[z % 2],
                                  agg_out.at[cid, pl.ds(cix * K, K)],
                                  ssems[z % 2]).wait()
    pltpu.sync_copy(deg_sp.at[pl.ds(dbase, DPT)], zero1d)
    pltpu.sync_copy(zero1d, deg_out.at[cid, pl.ds(dbase, DPT)])

    plsc.subcore_barrier()

    for ci in range(IC):
        pltpu.async_copy(pres_sp.at[ids_vm.at[ci]],
                         rk_vm.at[pl.ds(ci * IK, IK)], semG0).wait()
    off = cid * N
    for ci in range(IC):
        for s8 in range(IK // LANES):
            rv = rk_vm[pl.ds(ci * IK + s8 * LANES, LANES)]
            idx_vm[ci, pl.ds(s8 * LANES, LANES)] = rv.astype(jnp.int32) + off
    pltpu.sync_copy(idx_vm, idx_out.at[cid, sid])


def _tc_body(x_ref, agg_ref, deg_ref, ws_ref, wn_ref, wh_ref, bs_ref, bn_ref,
             p_ref):
    x = x_ref[...]
    a = agg_ref[0]
    dg = deg_ref[0, 0].reshape(RB, 1)
    mean = a / jnp.maximum(dg, 1.0)
    t = jnp.dot(x, ws_ref[...], preferred_element_type=jnp.float32)
    t = t + jnp.dot(mean, wn_ref[...], preferred_element_type=jnp.float32)
    t = t + bs_ref[...] + bn_ref[...]
    h = jnp.maximum(t, 0.0)
    nrm = jnp.sqrt(jnp.sum(h * h, axis=1, keepdims=True))
    s = h / jnp.maximum(nrm, 1e-12)
    p_ref[0] = jnp.dot(s, wh_ref[0], preferred_element_type=jnp.float32)


def _stage_b(x2, agg, deg3, w_self, w_neigh, wh, bs, bn):
    return pl.pallas_call(
        _tc_body,
        grid=(NC, GR),
        in_specs=[
            pl.BlockSpec((RB, D), lambda g, r: (g * GR + r, 0)),
            pl.BlockSpec((1, RB, D), lambda g, r: (g, r, 0)),
            pl.BlockSpec((1, 1, RB), lambda g, r: (g * GR + r, 0, 0)),
            pl.BlockSpec((D, H), lambda g, r: (0, 0)),
            pl.BlockSpec((D, H), lambda g, r: (0, 0)),
            pl.BlockSpec((1, H, O), lambda g, r: (g, 0, 0)),
            pl.BlockSpec((1, H), lambda g, r: (0, 0)),
            pl.BlockSpec((1, H), lambda g, r: (0, 0)),
        ],
        out_specs=pl.BlockSpec((1, RB, O), lambda g, r: (g, r, 0)),
        out_shape=jax.ShapeDtypeStruct((NC, N, O), jnp.float32),
    )(x2, agg, deg3, w_self, w_neigh, wh, bs, bn)


@functools.partial(
    pl.kernel,
    out_type=jax.ShapeDtypeStruct((B, O), jnp.float32),
    mesh=_mesh,
    scratch_types=[
        pltpu.VMEM((CPT // IK, IK), jnp.int32),
        pltpu.VMEM((CPT // IK, IK), jnp.int32),
        pltpu.VMEM((CPT, O), jnp.float32),
        pltpu.VMEM((CPT, O), jnp.float32),
        pltpu.VMEM((O,), jnp.float32),
        pltpu.SemaphoreType.DMA,
    ],
    compiler_params=pltpu.CompilerParams(needs_layout_passes=False),
)
def _stage_c(p2, idxr, b_out, out, idxA, idxB, rowsA, rowsB, bvm, sem):
    cid = lax.axis_index("c")
    sid = lax.axis_index("s")
    w = cid * NS + sid

    pltpu.sync_copy(idxr.at[0, w], idxA)
    pltpu.sync_copy(idxr.at[1, w], idxB)
    pltpu.sync_copy(b_out, bvm)
    for cc in range(CPT // IK):
        pltpu.async_copy(p2.at[idxA.at[cc]],
                         rowsA.at[pl.ds(cc * IK, IK)], sem).wait()
        pltpu.async_copy(p2.at[idxB.at[cc]],
                         rowsB.at[pl.ds(cc * IK, IK)], sem).wait()

    def _rbody(r, carry):
        for c8 in range(O // LANES):
            sl = pl.ds(c8 * LANES, LANES)
            rowsA[r, sl] = rowsA[r, sl] + rowsB[r, sl] + bvm[sl]
        return carry
    lax.fori_loop(0, CPT, _rbody, 0)

    pltpu.sync_copy(rowsA, out.at[pl.ds(w * CPT, CPT)])


def kernel(x_i, edge_index_i, node_ids_i, x_j, edge_index_j, node_ids_j,
           W_self, b_self, W_neigh, b_neigh, W_out, b_out):
    x_i = x_i.astype(jnp.float32)
    x_j = x_j.astype(jnp.float32)
    x2 = jnp.concatenate([x_i, x_j], axis=0)
    ei = edge_index_i.astype(jnp.int32)
    ej = edge_index_j.astype(jnp.int32)
    e_i = ei.reshape(2, NS, NBLK, EB, K)
    e_j = jnp.stack([ej[0] + N, ej[1]]).reshape(2, NS, NBLK, EB, K)
    edges = jnp.stack([e_i, e_j]).transpose(0, 2, 3, 1, 4, 5)
    ids = jnp.stack([node_ids_i.astype(jnp.int32).reshape(NS, IC, IK),
                     node_ids_j.astype(jnp.int32).reshape(NS, IC, IK)])

    agg, deg, idx = _stage_a(x2, edges, ids)
    deg3 = deg[:, :N].reshape(NC * GR, 1, RB)
    bs = b_self.astype(jnp.float32).reshape(1, H)
    bn = b_neigh.astype(jnp.float32).reshape(1, H)
    wh = W_out.astype(jnp.float32).reshape(NC, H, O)
    p = _stage_b(x2, agg, deg3, W_self.astype(jnp.float32),
                 W_neigh.astype(jnp.float32), wh, bs, bn)

    p2 = p.reshape(NC * N, O)
    idxr = idx.reshape(NC, NC * NS, CPT // IK, IK)
    h = _stage_c(p2, idxr, b_out.astype(jnp.float32))
    return h[None]


# speedup vs baseline: 1.0947x; 1.0365x over previous
import functools

import jax
import jax.numpy as jnp
from jax import lax
from jax.experimental import pallas as pl
from jax.experimental.pallas import tpu as pltpu
from jax.experimental.pallas import tpu_sc as plsc

N, E, D, H, O, B = 10000, 320000, 128, 128, 128, 8192
NC, NS, LANES = 2, 16, 16
K = 80
EB = 25
NBLK = E // NS // K // EB
NPAD = 10240
DPT = NPAD // NS
APT = N // NS
IC, IK = 4, 128
RB = 2000
GR = N // RB
CPT = B // (NC * NS)

_mesh = plsc.VectorSubcoreMesh(core_axis_name="c", subcore_axis_name="s")


@functools.partial(
    pl.kernel,
    out_type=(
        jax.ShapeDtypeStruct((NC, N, D), jnp.float32),
        jax.ShapeDtypeStruct((NC, NPAD), jnp.float32),
        jax.ShapeDtypeStruct((NC, NS, IC, IK), jnp.int32),
    ),
    mesh=_mesh,
    scratch_types=[
        pltpu.VMEM((K, D), jnp.float32),
        pltpu.VMEM((K, D), jnp.float32),
        pltpu.VMEM((K, D), jnp.float32),
        pltpu.VMEM((2, EB, K), jnp.int32),
        pltpu.VMEM((IC, IK), jnp.int32),
        pltpu.VMEM((IC, IK), jnp.int32),
        pltpu.VMEM((IC * IK,), jnp.float32),
        pltpu.VMEM((DPT,), jnp.float32),
        pltpu.VMEM((IK,), jnp.float32),
        pltpu.VMEM_SHARED((N, D), jnp.float32),
        pltpu.VMEM_SHARED((NPAD,), jnp.float32),
        pltpu.VMEM_SHARED((NPAD,), jnp.float32),
        pltpu.VMEM_SHARED((NS * LANES,), jnp.float32),
        pltpu.SemaphoreType.DMA,
        pltpu.SemaphoreType.DMA,
        pltpu.SemaphoreType.DMA,
        pltpu.SemaphoreType.DMA,
        pltpu.SemaphoreType.DMA,
        pltpu.SemaphoreType.DMA,
        pltpu.SemaphoreType.DMA,
    ],
    compiler_params=pltpu.CompilerParams(needs_layout_passes=False),
)
def _stage_a(x2, edges, ids,
             agg_out, deg_out, idx_out,
             buf0, buf1, buf2, ebuf, ids_vm, idx_vm, rk_vm,
             zero1d, ones_vm, agg_sp, deg_sp, pres_sp, tot_sp,
             semG0, semG1, semG2, semS0, semS1, semS2, semD):
    cid = lax.axis_index("c")
    sid = lax.axis_index("s")
    dbase = sid * DPT
    abase = sid * APT
    bufs = (buf0, buf1, buf2)
    gsems = (semG0, semG1, semG2)
    ssems = (semS0, semS1, semS2)

    z16 = jnp.zeros((LANES,), jnp.float32)
    o16 = jnp.ones((LANES,), jnp.float32)

    def _zrow(r, carry):
        for c in range(D // LANES):
            buf0[r, pl.ds(c * LANES, LANES)] = z16
        return carry
    lax.fori_loop(0, K, _zrow, 0)

    def _z1d(k, carry):
        zero1d[pl.ds(k * LANES, LANES)] = z16
        return carry
    lax.fori_loop(0, DPT // LANES, _z1d, 0)

    for c in range(IK // LANES):
        ones_vm[pl.ds(c * LANES, LANES)] = o16

    for z in range(7):
        pltpu.async_copy(buf0, agg_sp.at[pl.ds(abase + z * K, K)], ssems[0])
    pltpu.async_copy(buf0.at[pl.ds(0, APT - 7 * K)],
                     agg_sp.at[pl.ds(abase + 7 * K, APT - 7 * K)], ssems[1])
    pltpu.async_copy(zero1d, deg_sp.at[pl.ds(dbase, DPT)], ssems[2])
    pltpu.async_copy(zero1d, pres_sp.at[pl.ds(dbase, DPT)], semD)
    for z in range(7):
        pltpu.make_async_copy(buf0, agg_sp.at[pl.ds(abase + z * K, K)],
                              ssems[0]).wait()
    pltpu.make_async_copy(buf0.at[pl.ds(0, APT - 7 * K)],
                          agg_sp.at[pl.ds(abase + 7 * K, APT - 7 * K)],
                          ssems[1]).wait()
    pltpu.make_async_copy(zero1d, deg_sp.at[pl.ds(dbase, DPT)],
                          ssems[2]).wait()
    pltpu.make_async_copy(zero1d, pres_sp.at[pl.ds(dbase, DPT)],
                          semD).wait()

    pltpu.sync_copy(ids.at[cid, sid], ids_vm)

    plsc.subcore_barrier()

    for ci in range(IC):
        pltpu.sync_copy(ones_vm, pres_sp.at[ids_vm.at[ci]], add=True)

    plsc.subcore_barrier()

    def _eblk(b, carry):
        pltpu.sync_copy(edges.at[cid, sid, b], ebuf)
        for c in range(3):
            pltpu.async_copy(x2.at[ebuf.at[0, c]], bufs[c], gsems[c])
        for c in range(EB):
            i3 = c % 3
            pltpu.make_async_copy(x2.at[ebuf.at[0, c]], bufs[i3],
                                  gsems[i3]).wait()
            pltpu.async_copy(bufs[i3], agg_sp.at[ebuf.at[1, c]], ssems[i3],
                             add=True)
            pltpu.async_copy(ones_vm.at[pl.ds(0, K)],
                             deg_sp.at[ebuf.at[1, c]], semD, add=True)
            if c >= 1 and c + 2 < EB:
                j3 = (c - 1) % 3
                pltpu.make_async_copy(bufs[j3], agg_sp.at[ebuf.at[1, c - 1]],
                                      ssems[j3]).wait()
                pltpu.async_copy(x2.at[ebuf.at[0, c + 2]], bufs[j3],
                                 gsems[j3])
        for c in (EB - 3, EB - 2, EB - 1):
            pltpu.make_async_copy(bufs[c % 3], agg_sp.at[ebuf.at[1, c]],
                                  ssems[c % 3]).wait()
        for c in range(EB):
            pltpu.make_async_copy(ones_vm.at[pl.ds(0, K)],
                                  deg_sp.at[ebuf.at[1, c]], semD).wait()
        return carry
    lax.fori_loop(0, NBLK, _eblk, 0)

    pltpu.sync_copy(pres_sp.at[pl.ds(dbase, DPT)], zero1d)

    def _pbody(k, cin):
        v = zero1d[pl.ds(k * LANES, LANES)]
        p1 = jnp.minimum(v, 1.0)
        cs = plsc.cumsum(p1)
        zero1d[pl.ds(k * LANES, LANES)] = cin + cs - p1
        return cin + cs[LANES - 1]
    total = lax.fori_loop(0, DPT // LANES, _pbody, jnp.float32(0.0))
    rk_vm[pl.ds(0, LANES)] = jnp.zeros((LANES,), jnp.float32) + total
    pltpu.sync_copy(rk_vm.at[pl.ds(0, LANES)],
                    tot_sp.at[pl.ds(sid * LANES, LANES)])

    plsc.subcore_barrier()

    pltpu.sync_copy(tot_sp, rk_vm.at[pl.ds(0, NS * LANES)])
    lane = lax.iota(jnp.int32, LANES)
    tots = plsc.load_gather(rk_vm, [lane * LANES])
    ex = plsc.cumsum(tots) - tots
    sel = jnp.where(lane == sid, ex, 0.0)
    off = plsc.cumsum(sel)[LANES - 1]

    def _obody(k, carry):
        sl = pl.ds(k * LANES, LANES)
        zero1d[sl] = zero1d[sl] + off
        return carry
    lax.fori_loop(0, DPT // LANES, _obody, 0)
    pltpu.sync_copy(zero1d, pres_sp.at[pl.ds(dbase, DPT)])

    _nz = N // K // NS + 1
    for z in range(_nz):
        cix = z * NS + sid

        @pl.when(cix < N // K)
        def _(z=z, cix=cix):
            i2 = z % 2
            if z >= 2:
                pc = (z - 2) * NS + sid
                pltpu.make_async_copy(bufs[i2],
                                      agg_out.at[cid, pl.ds(pc * K, K)],
                                      ssems[i2]).wait()
            pltpu.async_copy(agg_sp.at[pl.ds(cix * K, K)], bufs[i2],
                             gsems[i2]).wait()
            pltpu.async_copy(bufs[i2], agg_out.at[cid, pl.ds(cix * K, K)],
                             ssems[i2])
    for z in (_nz - 2, _nz - 1):
        cix = z * NS + sid

        @pl.when(cix < N // K)
        def _(z=z, cix=cix):
            pltpu.make_async_copy(bufs[z % 2],
                                  agg_out.at[cid, pl.ds(cix * K, K)],
                                  ssems[z % 2]).wait()
    pltpu.sync_copy(deg_sp.at[pl.ds(dbase, DPT)], zero1d)
    pltpu.sync_copy(zero1d, deg_out.at[cid, pl.ds(dbase, DPT)])

    plsc.subcore_barrier()

    for ci in range(IC):
        pltpu.async_copy(pres_sp.at[ids_vm.at[ci]],
                         rk_vm.at[pl.ds(ci * IK, IK)], semG0).wait()
    off = cid * N
    for ci in range(IC):
        for s8 in range(IK // LANES):
            rv = rk_vm[pl.ds(ci * IK + s8 * LANES, LANES)]
            idx_vm[ci, pl.ds(s8 * LANES, LANES)] = rv.astype(jnp.int32) + off
    pltpu.sync_copy(idx_vm, idx_out.at[cid, sid])


def _tc_body(x_ref, agg_ref, deg_ref, ws_ref, wn_ref, wh_ref, bs_ref, bn_ref,
             bo_ref, p_ref):
    x = x_ref[...]
    a = agg_ref[0]
    dg = deg_ref[0, 0].reshape(RB, 1)
    mean = a / jnp.maximum(dg, 1.0)
    t = jnp.dot(x, ws_ref[...], preferred_element_type=jnp.float32)
    t = t + jnp.dot(mean, wn_ref[...], preferred_element_type=jnp.float32)
    t = t + bs_ref[...] + bn_ref[...]
    h = jnp.maximum(t, 0.0)
    nrm = jnp.sqrt(jnp.sum(h * h, axis=1, keepdims=True))
    s = h / jnp.maximum(nrm, 1e-12)
    gmask = (pl.program_id(0) == 1).astype(jnp.float32)
    p_ref[0] = (jnp.dot(s, wh_ref[0], preferred_element_type=jnp.float32)
                + gmask * bo_ref[...])


def _stage_b(x2, agg, deg3, w_self, w_neigh, wh, bs, bn, bo):
    return pl.pallas_call(
        _tc_body,
        grid=(NC, GR),
        in_specs=[
            pl.BlockSpec((RB, D), lambda g, r: (g * GR + r, 0)),
            pl.BlockSpec((1, RB, D), lambda g, r: (g, r, 0)),
            pl.BlockSpec((1, 1, RB), lambda g, r: (g * GR + r, 0, 0)),
            pl.BlockSpec((D, H), lambda g, r: (0, 0)),
            pl.BlockSpec((D, H), lambda g, r: (0, 0)),
            pl.BlockSpec((1, H, O), lambda g, r: (g, 0, 0)),
            pl.BlockSpec((1, H), lambda g, r: (0, 0)),
            pl.BlockSpec((1, H), lambda g, r: (0, 0)),
            pl.BlockSpec((1, O), lambda g, r: (0, 0)),
        ],
        out_specs=pl.BlockSpec((1, RB, O), lambda g, r: (g, r, 0)),
        out_shape=jax.ShapeDtypeStruct((NC, N, O), jnp.float32),
    )(x2, agg, deg3, w_self, w_neigh, wh, bs, bn, bo)


@functools.partial(
    pl.kernel,
    out_type=jax.ShapeDtypeStruct((B, O), jnp.float32),
    mesh=_mesh,
    scratch_types=[
        pltpu.VMEM((CPT // IK, IK), jnp.int32),
        pltpu.VMEM((CPT // IK, IK), jnp.int32),
        pltpu.VMEM((CPT, O), jnp.float32),
        pltpu.VMEM((CPT, O), jnp.float32),
        pltpu.SemaphoreType.DMA,
        pltpu.SemaphoreType.DMA,
    ],
    compiler_params=pltpu.CompilerParams(needs_layout_passes=False),
)
def _stage_c(p2, idxr, out, idxA, idxB, rowsA, rowsB, semA, semB):
    cid = lax.axis_index("c")
    sid = lax.axis_index("s")
    w = cid * NS + sid

    pltpu.sync_copy(idxr.at[0, w], idxA)
    pltpu.sync_copy(idxr.at[1, w], idxB)
    for cc in range(CPT // IK):
        pltpu.async_copy(p2.at[idxA.at[cc]],
                         rowsA.at[pl.ds(cc * IK, IK)], semA)
        pltpu.async_copy(p2.at[idxB.at[cc]],
                         rowsB.at[pl.ds(cc * IK, IK)], semB)
    for cc in range(CPT // IK):
        pltpu.make_async_copy(p2.at[idxA.at[cc]],
                              rowsA.at[pl.ds(cc * IK, IK)], semA).wait()
        pltpu.make_async_copy(p2.at[idxB.at[cc]],
                              rowsB.at[pl.ds(cc * IK, IK)], semB).wait()

    def _rbody(r, carry):
        for c8 in range(O // LANES):
            sl = pl.ds(c8 * LANES, LANES)
            rowsA[r, sl] = rowsA[r, sl] + rowsB[r, sl]
        return carry
    lax.fori_loop(0, CPT, _rbody, 0)

    pltpu.sync_copy(rowsA, out.at[pl.ds(w * CPT, CPT)])


def kernel(x_i, edge_index_i, node_ids_i, x_j, edge_index_j, node_ids_j,
           W_self, b_self, W_neigh, b_neigh, W_out, b_out):
    x_i = x_i.astype(jnp.float32)
    x_j = x_j.astype(jnp.float32)
    x2 = jnp.concatenate([x_i, x_j], axis=0)
    ei = edge_index_i.astype(jnp.int32)
    ej = edge_index_j.astype(jnp.int32)
    e_i = ei.reshape(2, NS, NBLK, EB, K)
    e_j = jnp.stack([ej[0] + N, ej[1]]).reshape(2, NS, NBLK, EB, K)
    edges = jnp.stack([e_i, e_j]).transpose(0, 2, 3, 1, 4, 5)
    ids = jnp.stack([node_ids_i.astype(jnp.int32).reshape(NS, IC, IK),
                     node_ids_j.astype(jnp.int32).reshape(NS, IC, IK)])

    agg, deg, idx = _stage_a(x2, edges, ids)
    deg3 = deg[:, :N].reshape(NC * GR, 1, RB)
    bs = b_self.astype(jnp.float32).reshape(1, H)
    bn = b_neigh.astype(jnp.float32).reshape(1, H)
    wh = W_out.astype(jnp.float32).reshape(NC, H, O)
    p = _stage_b(x2, agg, deg3, W_self.astype(jnp.float32),
                 W_neigh.astype(jnp.float32), wh, bs, bn,
                 b_out.astype(jnp.float32).reshape(1, O))

    p2 = p.reshape(NC * N, O)
    idxr = idx.reshape(NC, NC * NS, CPT // IK, IK)
    h = _stage_c(p2, idxr)
    return h[None]
